# baseline (speedup 1.0000x reference)
import functools

import jax
import jax.numpy as jnp
from jax import lax
from jax.experimental import pallas as pl
from jax.experimental.pallas import tpu as pltpu
from jax.experimental.pallas import tpu_sc as plsc

NC = 2
NS = 16
NW = NC * NS
K = 96
NCH = 54
E_PAD = NW * NCH * K
ZROWS = 64


def _matmul_body(x_ref, w_ref, xa_ref, xb_ref):
    p = jnp.dot(x_ref[...], w_ref[...], preferred_element_type=jnp.float32)
    xa_ref[...] = p[:, :128]
    xb_ref[...] = p[:, 128:]


def _combine_body(x_ref, w0_ref, aggA_ref, aggB_ref, deg_ref, out_ref):
    out0 = jnp.dot(x_ref[...], w0_ref[...], preferred_element_type=jnp.float32)
    a = aggA_ref[0] + aggA_ref[1]
    b = aggB_ref[0] + aggB_ref[1]
    d = jnp.sum(deg_ref[...], axis=0)[:, None]
    r = 1.0 / jnp.maximum(d, 1.0)
    out_ref[...] = (out0 + jnp.concatenate([a, b], axis=1)) * r


def _sc_body(xa, xb, srcb, dstb, zhbm, aggA, aggB, degout,
             src_v, dst_v, rows0, rows1, deg_v, acc, sem_g0, sem_g1):
    rows = (rows0, rows1)
    sem_g = (sem_g0, sem_g1)
    c = lax.axis_index("c")
    s = lax.axis_index("s")
    w = c * NS + s
    n_acc = acc.shape[0]
    rows_per_tile = n_acc // NS
    base = s * rows_per_tile

    with jax.named_scope("sc_init"):
        pltpu.sync_copy(srcb.at[w], src_v)
        pltpu.sync_copy(dstb.at[w], dst_v)

        def _init_deg(i, _):
            deg_v[pl.ds(i * 16, 16)] = jnp.zeros((16,), jnp.float32)
            return _
        lax.fori_loop(0, n_acc // 16, _init_deg, 0)

    ones16 = jnp.ones((16,), jnp.float32)
    for p in range(2):
        xw = xa if p == 0 else xb
        agg = aggA if p == 0 else aggB

        with jax.named_scope(f"sc_zero{p}"):
            pltpu.async_copy(xw.at[src_v.at[0]], rows[0], sem_g[0])
            pltpu.sync_copy(zhbm.at[pl.ds(base, rows_per_tile)],
                            acc.at[pl.ds(base, rows_per_tile)])
            plsc.subcore_barrier()

        def _step(i, carry):
            for b in range(2):
                b1 = 1 - b
                j = 2 * i + b
                pltpu.make_async_copy(xw.at[src_v.at[j]], rows[b],
                                      sem_g[b]).wait()

                @pl.when(j + 1 < NCH)
                def _():
                    pltpu.async_copy(xw.at[src_v.at[j + 1]], rows[b1],
                                     sem_g[b1])

                if p == 0:
                    for v in range(K // 16):
                        idx = dst_v[j, pl.ds(v * 16, 16)]
                        plsc.addupdate_scatter(deg_v, [idx], ones16)
                pltpu.sync_copy(rows[b], acc.at[dst_v.at[j]], add=True)
            return carry
        with jax.named_scope(f"sc_chunks{p}"):
            lax.fori_loop(0, NCH // 2, _step, 0)
            plsc.subcore_barrier()

        with jax.named_scope(f"sc_copyout{p}"):
            pltpu.sync_copy(acc.at[pl.ds(base, rows_per_tile)],
                            agg.at[c].at[pl.ds(base, rows_per_tile)])
            plsc.subcore_barrier()

    pltpu.sync_copy(deg_v, degout.at[w])


def kernel(x, edge_index, num_nodes, W, W0):
    n = x.shape[0]
    d_in = x.shape[1]
    d_out = W.shape[1]
    e = edge_index.shape[1]
    rows_per_tile = -(-(n + 1) // (NS * ZROWS)) * ZROWS
    n_acc = rows_per_tile * NS

    src = edge_index[0]
    dst = edge_index[1]
    pad = E_PAD - e
    pad_src = (jnp.arange(pad, dtype=jnp.int32) * 37) % n
    n_dummy = n_acc - n
    pad_dst = n + (jnp.arange(pad, dtype=jnp.int32) % n_dummy)
    src_b = jnp.concatenate([src, pad_src]).reshape(NW, NCH, K)
    dst_b = jnp.concatenate([dst, pad_dst]).reshape(NW, NCH, K)

    bm = 1000
    xa, xb = pl.pallas_call(
        _matmul_body,
        grid=(n // bm,),
        in_specs=[
            pl.BlockSpec((bm, d_in), lambda i: (i, 0)),
            pl.BlockSpec((d_in, d_out), lambda i: (0, 0)),
        ],
        out_specs=[
            pl.BlockSpec((bm, 128), lambda i: (i, 0)),
            pl.BlockSpec((bm, 128), lambda i: (i, 0)),
        ],
        out_shape=[
            jax.ShapeDtypeStruct((n, 128), jnp.float32),
            jax.ShapeDtypeStruct((n, 128), jnp.float32),
        ],
    )(x, W)

    mesh = plsc.VectorSubcoreMesh(core_axis_name="c", subcore_axis_name="s")
    sc_call = pl.kernel(
        _sc_body,
        out_type=[
            jax.ShapeDtypeStruct((NC, n_acc, 128), jnp.float32),
            jax.ShapeDtypeStruct((NC, n_acc, 128), jnp.float32),
            jax.ShapeDtypeStruct((NW, n_acc), jnp.float32),
        ],
        mesh=mesh,
        scratch_types=[
            pltpu.VMEM((NCH, K), jnp.int32),
            pltpu.VMEM((NCH, K), jnp.int32),
            pltpu.VMEM((K, 128), jnp.float32),
            pltpu.VMEM((K, 128), jnp.float32),
            pltpu.VMEM((n_acc,), jnp.float32),
            pltpu.VMEM_SHARED((n_acc, 128), jnp.float32),
            pltpu.SemaphoreType.DMA,
            pltpu.SemaphoreType.DMA,
        ],
        compiler_params=pltpu.CompilerParams(needs_layout_passes=False),
    )
    zeros_hbm = jnp.zeros((n_acc, 128), jnp.float32)
    aggA, aggB, deg = sc_call(xa, xb, src_b, dst_b, zeros_hbm)

    bc = 1024
    gc = -(-n // bc)
    out = pl.pallas_call(
        _combine_body,
        grid=(gc,),
        in_specs=[
            pl.BlockSpec((bc, d_in), lambda i: (i, 0)),
            pl.BlockSpec((d_in, d_out), lambda i: (0, 0)),
            pl.BlockSpec((NC, bc, 128), lambda i: (0, i, 0)),
            pl.BlockSpec((NC, bc, 128), lambda i: (0, i, 0)),
            pl.BlockSpec((NW, bc), lambda i: (0, i)),
        ],
        out_specs=pl.BlockSpec((bc, d_out), lambda i: (i, 0)),
        out_shape=jax.ShapeDtypeStruct((n, d_out), jnp.float32),
    )(x, W0, aggA, aggB, deg)
    return out


# speedup vs baseline: 1.0214x; 1.0214x over previous
import functools

import jax
import jax.numpy as jnp
from jax import lax
from jax.experimental import pallas as pl
from jax.experimental.pallas import tpu as pltpu
from jax.experimental.pallas import tpu_sc as plsc

NC = 2
NS = 16
NW = NC * NS
K = 96
NCH = 54
E_PAD = NW * NCH * K
ZROWS = 64


def _matmul_body(x_ref, w_ref, xa_ref, xb_ref):
    p = jnp.dot(x_ref[...], w_ref[...], preferred_element_type=jnp.float32)
    xa_ref[...] = p[:, :128]
    xb_ref[...] = p[:, 128:]


def _matmul0_body(x_ref, w0_ref, out_ref):
    out_ref[...] = jnp.dot(x_ref[...], w0_ref[...],
                           preferred_element_type=jnp.float32)


def _combine_body(xw0_ref, aggA_ref, aggB_ref, deg_ref, out_ref):
    a = aggA_ref[0] + aggA_ref[1]
    b = aggB_ref[0] + aggB_ref[1]
    d = jnp.sum(deg_ref[...], axis=0)[:, None]
    r = 1.0 / jnp.maximum(d, 1.0)
    out_ref[...] = (xw0_ref[...] + jnp.concatenate([a, b], axis=1)) * r


def _sc_body(xa, xb, srcb, dstb, zhbm, aggA, aggB, degout,
             src_v, dst_v, rows0, rows1, deg_v, acc, sem_g0, sem_g1):
    rows = (rows0, rows1)
    sem_g = (sem_g0, sem_g1)
    c = lax.axis_index("c")
    s = lax.axis_index("s")
    w = c * NS + s
    n_acc = acc.shape[0]
    rows_per_tile = n_acc // NS
    base = s * rows_per_tile

    with jax.named_scope("sc_init"):
        pltpu.sync_copy(srcb.at[w], src_v)
        pltpu.sync_copy(dstb.at[w], dst_v)

        def _init_deg(i, _):
            deg_v[pl.ds(i * 16, 16)] = jnp.zeros((16,), jnp.float32)
            return _
        lax.fori_loop(0, n_acc // 16, _init_deg, 0)

    ones16 = jnp.ones((16,), jnp.float32)

    def _make_step(xw, with_deg):
        def _step(i, carry):
            for b in range(2):
                b1 = 1 - b
                j = 2 * i + b
                pltpu.make_async_copy(xw.at[src_v.at[j]], rows[b],
                                      sem_g[b]).wait()

                @pl.when(j + 1 < NCH)
                def _():
                    pltpu.async_copy(xw.at[src_v.at[j + 1]], rows[b1],
                                     sem_g[b1])

                if with_deg:
                    for v in range(K // 16):
                        idx = dst_v[j, pl.ds(v * 16, 16)]
                        plsc.addupdate_scatter(deg_v, [idx], ones16)
                pltpu.sync_copy(rows[b], acc.at[dst_v.at[j]], add=True)
            return carry
        return _step

    with jax.named_scope("sc_zero0"):
        pltpu.async_copy(xa.at[src_v.at[0]], rows[0], sem_g[0])
        pltpu.sync_copy(zhbm.at[pl.ds(base, rows_per_tile)],
                        acc.at[pl.ds(base, rows_per_tile)])
        plsc.subcore_barrier()
    with jax.named_scope("sc_chunks0"):
        lax.fori_loop(0, NCH // 2, _make_step(xa, True), 0)
        plsc.subcore_barrier()
    with jax.named_scope("sc_turn"):
        pltpu.async_copy(xb.at[src_v.at[0]], rows[0], sem_g[0])
        pltpu.sync_copy(acc.at[pl.ds(base, rows_per_tile)],
                        aggA.at[c].at[pl.ds(base, rows_per_tile)])
        pltpu.sync_copy(zhbm.at[pl.ds(base, rows_per_tile)],
                        acc.at[pl.ds(base, rows_per_tile)])
        plsc.subcore_barrier()
    with jax.named_scope("sc_chunks1"):
        lax.fori_loop(0, NCH // 2, _make_step(xb, False), 0)
        plsc.subcore_barrier()
    with jax.named_scope("sc_out1"):
        pltpu.sync_copy(acc.at[pl.ds(base, rows_per_tile)],
                        aggB.at[c].at[pl.ds(base, rows_per_tile)])

    pltpu.sync_copy(deg_v, degout.at[w])


def kernel(x, edge_index, num_nodes, W, W0):
    n = x.shape[0]
    d_in = x.shape[1]
    d_out = W.shape[1]
    e = edge_index.shape[1]
    rows_per_tile = -(-(n + 1) // (NS * ZROWS)) * ZROWS
    n_acc = rows_per_tile * NS

    src = edge_index[0]
    dst = edge_index[1]
    pad = E_PAD - e
    pad_src = (jnp.arange(pad, dtype=jnp.int32) * 37) % n
    n_dummy = n_acc - n
    pad_dst = n + (jnp.arange(pad, dtype=jnp.int32) % n_dummy)
    src_b = jnp.concatenate([src, pad_src]).reshape(NW, NCH, K)
    dst_b = jnp.concatenate([dst, pad_dst]).reshape(NW, NCH, K)

    bm = 1000
    xa, xb = pl.pallas_call(
        _matmul_body,
        grid=(n // bm,),
        in_specs=[
            pl.BlockSpec((bm, d_in), lambda i: (i, 0)),
            pl.BlockSpec((d_in, d_out), lambda i: (0, 0)),
        ],
        out_specs=[
            pl.BlockSpec((bm, 128), lambda i: (i, 0)),
            pl.BlockSpec((bm, 128), lambda i: (i, 0)),
        ],
        out_shape=[
            jax.ShapeDtypeStruct((n, 128), jnp.float32),
            jax.ShapeDtypeStruct((n, 128), jnp.float32),
        ],
    )(x, W)

    mesh = plsc.VectorSubcoreMesh(core_axis_name="c", subcore_axis_name="s")
    sc_call = pl.kernel(
        _sc_body,
        out_type=[
            jax.ShapeDtypeStruct((NC, n_acc, 128), jnp.float32),
            jax.ShapeDtypeStruct((NC, n_acc, 128), jnp.float32),
            jax.ShapeDtypeStruct((NW, n_acc), jnp.float32),
        ],
        mesh=mesh,
        scratch_types=[
            pltpu.VMEM((NCH, K), jnp.int32),
            pltpu.VMEM((NCH, K), jnp.int32),
            pltpu.VMEM((K, 128), jnp.float32),
            pltpu.VMEM((K, 128), jnp.float32),
            pltpu.VMEM((n_acc,), jnp.float32),
            pltpu.VMEM_SHARED((n_acc, 128), jnp.float32),
            pltpu.SemaphoreType.DMA,
            pltpu.SemaphoreType.DMA,
        ],
        compiler_params=pltpu.CompilerParams(needs_layout_passes=False),
    )
    zeros_hbm = jnp.zeros((n_acc, 128), jnp.float32)
    aggA, aggB, deg = sc_call(xa, xb, src_b, dst_b, zeros_hbm)

    xw0 = pl.pallas_call(
        _matmul0_body,
        grid=(n // bm,),
        in_specs=[
            pl.BlockSpec((bm, d_in), lambda i: (i, 0)),
            pl.BlockSpec((d_in, d_out), lambda i: (0, 0)),
        ],
        out_specs=pl.BlockSpec((bm, d_out), lambda i: (i, 0)),
        out_shape=jax.ShapeDtypeStruct((n, d_out), jnp.float32),
    )(x, W0)

    bc = 1024
    gc = -(-n // bc)
    out = pl.pallas_call(
        _combine_body,
        grid=(gc,),
        in_specs=[
            pl.BlockSpec((bc, d_out), lambda i: (i, 0)),
            pl.BlockSpec((NC, bc, 128), lambda i: (0, i, 0)),
            pl.BlockSpec((NC, bc, 128), lambda i: (0, i, 0)),
            pl.BlockSpec((NW, bc), lambda i: (0, i)),
        ],
        out_specs=pl.BlockSpec((bc, d_out), lambda i: (i, 0)),
        out_shape=jax.ShapeDtypeStruct((n, d_out), jnp.float32),
    )(xw0, aggA, aggB, deg)
    return out
